# baseline (speedup 1.0000x reference)
import functools

import jax
import jax.numpy as jnp
import numpy as np
from jax import lax
from jax.experimental import pallas as pl
from jax.experimental.pallas import tpu as pltpu
from jax.experimental.pallas import tpu_sc as plsc

B = 200000
F_DIM = 128
E_DIM = 64
OUT_DIM = F_DIM + E_DIM

NW = 32
CHUNK = 256
BP = B + 64
NCHUNKS = -(-BP // CHUNK)
LAST_BASE = BP - CHUNK
CPW = -(-NCHUNKS // NW)
GSUB = 128

TBS = 16384
TGRID = -(-B // TBS)


def _sc_gather_t(ids_p, emb128):
    mesh = plsc.VectorSubcoreMesh(core_axis_name="c", subcore_axis_name="s")

    @functools.partial(
        pl.kernel,
        mesh=mesh,
        out_type=jax.ShapeDtypeStruct((E_DIM, BP), jnp.float32),
        scratch_types=[
            pltpu.VMEM((2, CHUNK), jnp.int32),
            pltpu.VMEM((2, CHUNK, F_DIM), jnp.float32),
            pltpu.VMEM((E_DIM, CHUNK), jnp.float32),
            pltpu.SemaphoreType.DMA,
            pltpu.SemaphoreType.DMA,
            pltpu.SemaphoreType.DMA,
        ],
        compiler_params=pltpu.CompilerParams(needs_layout_passes=False),
    )
    def k(ids_hbm, emb_hbm, embt_hbm, idx_v, rows_v, t_v, sem_g0, sem_g1,
          sem_w):
        wid = lax.axis_index("s") * 2 + lax.axis_index("c")
        iota16 = lax.iota(jnp.int32, 16)

        def chunk_base(ci):
            return lax.min(ci * CHUNK, jnp.int32(LAST_BASE))

        def issue(ci, b, sem):
            base = chunk_base(ci)
            pltpu.sync_copy(ids_hbm.at[pl.ds(base, CHUNK)], idx_v.at[b])
            for s in range(0, CHUNK, GSUB):
                pltpu.async_copy(
                    emb_hbm.at[idx_v.at[b].at[pl.ds(s, GSUB)]],
                    rows_v.at[b].at[pl.ds(s, GSUB)], sem)

        def drain(b, sem):
            for s in range(0, CHUNK, GSUB):
                pltpu.make_async_copy(
                    emb_hbm.at[idx_v.at[b].at[pl.ds(s, GSUB)]],
                    rows_v.at[b].at[pl.ds(s, GSUB)], sem).wait()

        issue(wid, 0, sem_g0)

        def step(i, _):
            ci = wid + i * NW
            b = lax.rem(i, 2)

            @pl.when(wid + (i + 1) * NW < NCHUNKS)
            def _():
                @pl.when(b == 0)
                def _():
                    issue(wid + (i + 1) * NW, 1, sem_g1)

                @pl.when(b == 1)
                def _():
                    issue(wid + (i + 1) * NW, 0, sem_g0)

            @pl.when(ci < NCHUNKS)
            def _():
                @pl.when(b == 0)
                def _():
                    drain(0, sem_g0)

                @pl.when(b == 1)
                def _():
                    drain(1, sem_g1)

                def jrow0(j, _):
                    colv = jnp.full((16,), j, jnp.int32)
                    for c0 in range(0, E_DIM, 16):
                        vals = plsc.load_gather(
                            rows_v.at[0], [colv, iota16 + c0])
                        plsc.store_scatter(t_v, [iota16 + c0, colv], vals)
                    return ()

                def jrow1(j, _):
                    colv = jnp.full((16,), j, jnp.int32)
                    for c0 in range(0, E_DIM, 16):
                        vals = plsc.load_gather(
                            rows_v.at[1], [colv, iota16 + c0])
                        plsc.store_scatter(t_v, [iota16 + c0, colv], vals)
                    return ()

                @pl.when(b == 0)
                def _():
                    lax.fori_loop(0, CHUNK, jrow0, ())

                @pl.when(b == 1)
                def _():
                    lax.fori_loop(0, CHUNK, jrow1, ())

                pltpu.async_copy(
                    t_v, embt_hbm.at[:, pl.ds(chunk_base(ci), CHUNK)],
                    sem_w).wait()
            return ()

        lax.fori_loop(0, CPW, step, ())

    return k(ids_p, emb128)


def _tc_feats_t(feats):
    def body(feats_ref, out_ref):
        out_ref[...] = feats_ref[...].T

    return pl.pallas_call(
        body,
        grid=(TGRID,),
        in_specs=[pl.BlockSpec((TBS, F_DIM), lambda i: (i, 0))],
        out_specs=pl.BlockSpec((F_DIM, TBS), lambda i: (0, i)),
        out_shape=jax.ShapeDtypeStruct((OUT_DIM, B), jnp.float32),
    )(feats)


def _tc_emb_copy(out_t, embt):
    def body(_, emb_ref, out_ref):
        out_ref[...] = emb_ref[...]

    return pl.pallas_call(
        body,
        grid=(TGRID,),
        in_specs=[
            pl.BlockSpec(memory_space=pl.ANY),
            pl.BlockSpec((E_DIM, TBS), lambda i: (0, i)),
        ],
        out_specs=pl.BlockSpec((E_DIM, TBS), lambda i: (2, i)),
        out_shape=jax.ShapeDtypeStruct((OUT_DIM, B), jnp.float32),
        input_output_aliases={0: 0},
    )(out_t, embt)


def kernel(ids, feats, hop_idx, emb_W):
    n_nodes = emb_W.shape[0] - 1
    gather_ids = jnp.where(hop_idx > 0, ids,
                           jnp.full_like(ids, n_nodes)).astype(jnp.int32)
    ids_p = jnp.pad(gather_ids, (0, BP - B))
    emb128 = jnp.pad(emb_W, ((0, 0), (0, F_DIM - E_DIM)))
    embt = _sc_gather_t(ids_p, emb128)
    out_t = _tc_feats_t(feats)
    out_t = _tc_emb_copy(out_t, embt)
    return out_t.T


# speedup vs baseline: 1.3562x; 1.3562x over previous
import functools

import jax
import jax.numpy as jnp
import numpy as np
from jax import lax
from jax.experimental import pallas as pl
from jax.experimental.pallas import tpu as pltpu
from jax.experimental.pallas import tpu_sc as plsc

B = 200000
F_DIM = 128
E_DIM = 64
OUT_DIM = F_DIM + E_DIM

NW = 32
CHUNK = 256
BP = B + 64
NCHUNKS = -(-BP // CHUNK)
LAST_BASE = BP - CHUNK
CPW = -(-NCHUNKS // NW)
GSUB = 128

TBS = 16384
TGRID = -(-B // TBS)


def _sc_gather_t(ids_p, emb128):
    mesh = plsc.VectorSubcoreMesh(core_axis_name="c", subcore_axis_name="s")

    @functools.partial(
        pl.kernel,
        mesh=mesh,
        out_type=jax.ShapeDtypeStruct((E_DIM, BP), jnp.float32),
        scratch_types=[
            pltpu.VMEM((2, CHUNK), jnp.int32),
            pltpu.VMEM((2, CHUNK, F_DIM), jnp.float32),
            pltpu.VMEM((E_DIM, CHUNK), jnp.float32),
            pltpu.SemaphoreType.DMA,
            pltpu.SemaphoreType.DMA,
            pltpu.SemaphoreType.DMA,
        ],
        compiler_params=pltpu.CompilerParams(needs_layout_passes=False),
    )
    def k(ids_hbm, emb_hbm, embt_hbm, idx_v, rows_v, t_v, sem_g0, sem_g1,
          sem_w):
        wid = lax.axis_index("s") * 2 + lax.axis_index("c")
        iota16 = lax.iota(jnp.int32, 16)

        def chunk_base(ci):
            return lax.min(ci * CHUNK, jnp.int32(LAST_BASE))

        def issue(ci, b, sem):
            base = chunk_base(ci)
            pltpu.sync_copy(ids_hbm.at[pl.ds(base, CHUNK)], idx_v.at[b])
            for s in range(0, CHUNK, GSUB):
                pltpu.async_copy(
                    emb_hbm.at[idx_v.at[b].at[pl.ds(s, GSUB)]],
                    rows_v.at[b].at[pl.ds(s, GSUB)], sem)

        def drain(b, sem):
            for s in range(0, CHUNK, GSUB):
                pltpu.make_async_copy(
                    emb_hbm.at[idx_v.at[b].at[pl.ds(s, GSUB)]],
                    rows_v.at[b].at[pl.ds(s, GSUB)], sem).wait()

        issue(wid, 0, sem_g0)

        def step(i, _):
            ci = wid + i * NW
            b = lax.rem(i, 2)

            @pl.when(wid + (i + 1) * NW < NCHUNKS)
            def _():
                @pl.when(b == 0)
                def _():
                    issue(wid + (i + 1) * NW, 1, sem_g1)

                @pl.when(b == 1)
                def _():
                    issue(wid + (i + 1) * NW, 0, sem_g0)

            @pl.when(ci < NCHUNKS)
            def _():
                @pl.when(b == 0)
                def _():
                    drain(0, sem_g0)

                @pl.when(b == 1)
                def _():
                    drain(1, sem_g1)

                def make_jblock(rv):
                    def jblock(j0, _):
                        rowv = j0 * 16 + iota16
                        for c0 in range(0, E_DIM, 16):
                            for d in range(16):
                                colv = c0 + ((iota16 + d) & 15)
                                vals = plsc.load_gather(rv, [rowv, colv])
                                plsc.store_scatter(t_v, [colv, rowv], vals)
                        return ()
                    return jblock

                @pl.when(b == 0)
                def _():
                    lax.fori_loop(0, CHUNK // 16, make_jblock(rows_v.at[0]),
                                  ())

                @pl.when(b == 1)
                def _():
                    lax.fori_loop(0, CHUNK // 16, make_jblock(rows_v.at[1]),
                                  ())

                pltpu.async_copy(
                    t_v, embt_hbm.at[:, pl.ds(chunk_base(ci), CHUNK)],
                    sem_w).wait()
            return ()

        lax.fori_loop(0, CPW, step, ())

    return k(ids_p, emb128)


def _tc_feats_t(feats):
    def body(feats_ref, out_ref):
        out_ref[...] = feats_ref[...].T

    return pl.pallas_call(
        body,
        grid=(TGRID,),
        in_specs=[pl.BlockSpec((TBS, F_DIM), lambda i: (i, 0))],
        out_specs=pl.BlockSpec((F_DIM, TBS), lambda i: (0, i)),
        out_shape=jax.ShapeDtypeStruct((OUT_DIM, B), jnp.float32),
    )(feats)


def _tc_emb_copy(out_t, embt):
    def body(_, emb_ref, out_ref):
        out_ref[...] = emb_ref[...]

    return pl.pallas_call(
        body,
        grid=(TGRID,),
        in_specs=[
            pl.BlockSpec(memory_space=pl.ANY),
            pl.BlockSpec((E_DIM, TBS), lambda i: (0, i)),
        ],
        out_specs=pl.BlockSpec((E_DIM, TBS), lambda i: (2, i)),
        out_shape=jax.ShapeDtypeStruct((OUT_DIM, B), jnp.float32),
        input_output_aliases={0: 0},
    )(out_t, embt)


def kernel(ids, feats, hop_idx, emb_W):
    n_nodes = emb_W.shape[0] - 1
    gather_ids = jnp.where(hop_idx > 0, ids,
                           jnp.full_like(ids, n_nodes)).astype(jnp.int32)
    ids_p = jnp.pad(gather_ids, (0, BP - B))
    emb128 = jnp.pad(emb_W, ((0, 0), (0, F_DIM - E_DIM)))
    embt = _sc_gather_t(ids_p, emb128)
    out_t = _tc_feats_t(feats)
    out_t = _tc_emb_copy(out_t, embt)
    return out_t.T


# speedup vs baseline: 1.4935x; 1.1012x over previous
import functools

import jax
import jax.numpy as jnp
from jax import lax
from jax.experimental import pallas as pl
from jax.experimental.pallas import tpu as pltpu
from jax.experimental.pallas import tpu_sc as plsc

B = 200000
F_DIM = 128
E_DIM = 64
OUT_DIM = F_DIM + E_DIM

NW = 32
CHUNK = 320
NCHUNKS = B // CHUNK
CPW = -(-NCHUNKS // NW)
GSUB = 128

TBS = 16384
TGRID = -(-B // TBS)


def _sc_gather(ids, emb128):
    mesh = plsc.VectorSubcoreMesh(core_axis_name="c", subcore_axis_name="s")

    @functools.partial(
        pl.kernel,
        mesh=mesh,
        out_type=jax.ShapeDtypeStruct((B, F_DIM), jnp.float32),
        scratch_types=[
            pltpu.VMEM((CHUNK,), jnp.int32),
            pltpu.VMEM((CHUNK, F_DIM), jnp.float32),
            pltpu.SemaphoreType.DMA,
        ],
    )
    def k(ids_hbm, emb_hbm, wide_hbm, idx_v, rows_v, sem_g):
        wid = lax.axis_index("s") * 2 + lax.axis_index("c")

        def step(i, _):
            ci = wid + i * NW

            @pl.when(ci < NCHUNKS)
            def _():
                base = ci * CHUNK
                pltpu.sync_copy(ids_hbm.at[pl.ds(base, CHUNK)], idx_v)
                gathers = []
                for s in range(0, CHUNK, GSUB):
                    n = min(GSUB, CHUNK - s)
                    gathers.append(pltpu.async_copy(
                        emb_hbm.at[idx_v.at[pl.ds(s, n)]],
                        rows_v.at[pl.ds(s, n)], sem_g))
                for g in gathers:
                    g.wait()
                w_wide = pltpu.async_copy(
                    rows_v, wide_hbm.at[pl.ds(base, CHUNK), :], sem_g)
                w_wide.wait()
            return ()

        lax.fori_loop(0, CPW, step, ())

    return k(ids, emb128)


def _tc_feats_t(feats):
    def body(feats_ref, out_ref):
        out_ref[...] = feats_ref[...].T

    return pl.pallas_call(
        body,
        grid=(TGRID,),
        in_specs=[pl.BlockSpec((TBS, F_DIM), lambda i: (i, 0))],
        out_specs=pl.BlockSpec((F_DIM, TBS), lambda i: (0, i)),
        out_shape=jax.ShapeDtypeStruct((OUT_DIM, B), jnp.float32),
    )(feats)


def _tc_emb_t(out_t, wide):
    def body(_, wide_ref, out_ref):
        out_ref[...] = wide_ref[:, 0:E_DIM].T

    return pl.pallas_call(
        body,
        grid=(TGRID,),
        in_specs=[
            pl.BlockSpec(memory_space=pl.ANY),
            pl.BlockSpec((TBS, F_DIM), lambda i: (i, 0)),
        ],
        out_specs=pl.BlockSpec((E_DIM, TBS), lambda i: (2, i)),
        out_shape=jax.ShapeDtypeStruct((OUT_DIM, B), jnp.float32),
        input_output_aliases={0: 0},
    )(out_t, wide)


def kernel(ids, feats, hop_idx, emb_W):
    n_nodes = emb_W.shape[0] - 1
    gather_ids = jnp.where(hop_idx > 0, ids,
                           jnp.full_like(ids, n_nodes)).astype(jnp.int32)
    emb128 = jnp.pad(emb_W, ((0, 0), (0, F_DIM - E_DIM)))
    wide = _sc_gather(gather_ids, emb128)
    out_t = _tc_feats_t(feats)
    out_t = _tc_emb_t(out_t, wide)
    return out_t.T


# speedup vs baseline: 1.5028x; 1.0062x over previous
import functools

import jax
import jax.numpy as jnp
from jax import lax
from jax.experimental import pallas as pl
from jax.experimental.pallas import tpu as pltpu
from jax.experimental.pallas import tpu_sc as plsc

B = 200000
F_DIM = 128
E_DIM = 64
OUT_DIM = F_DIM + E_DIM

NW = 32
CHUNK = 320
NCHUNKS = B // CHUNK
CPW = -(-NCHUNKS // NW)
GSUB = 128

TBS = 16384
TGRID = -(-B // TBS)
CBS = 32768
CGRID = -(-B // CBS)


def _sc_gather(ids, emb128):
    mesh = plsc.VectorSubcoreMesh(core_axis_name="c", subcore_axis_name="s")

    @functools.partial(
        pl.kernel,
        mesh=mesh,
        out_type=jax.ShapeDtypeStruct((B, F_DIM), jnp.float32),
        scratch_types=[
            pltpu.VMEM((CHUNK,), jnp.int32),
            pltpu.VMEM((CHUNK, F_DIM), jnp.float32),
            pltpu.SemaphoreType.DMA,
        ],
    )
    def k(ids_hbm, emb_hbm, wide_hbm, idx_v, rows_v, sem_g):
        wid = lax.axis_index("s") * 2 + lax.axis_index("c")

        def step(i, _):
            ci = wid + i * NW

            @pl.when(ci < NCHUNKS)
            def _():
                base = ci * CHUNK
                pltpu.sync_copy(ids_hbm.at[pl.ds(base, CHUNK)], idx_v)
                gathers = []
                for s in range(0, CHUNK, GSUB):
                    n = min(GSUB, CHUNK - s)
                    gathers.append(pltpu.async_copy(
                        emb_hbm.at[idx_v.at[pl.ds(s, n)]],
                        rows_v.at[pl.ds(s, n)], sem_g))
                for g in gathers:
                    g.wait()
                w_wide = pltpu.async_copy(
                    rows_v, wide_hbm.at[pl.ds(base, CHUNK), :], sem_g)
                w_wide.wait()
            return ()

        lax.fori_loop(0, CPW, step, ())

    return k(ids, emb128)


def _tc_feats_t(feats):
    def body(feats_ref, out_ref):
        out_ref[...] = feats_ref[...].T

    return pl.pallas_call(
        body,
        grid=(TGRID,),
        in_specs=[pl.BlockSpec((TBS, F_DIM), lambda i: (i, 0))],
        out_specs=pl.BlockSpec((F_DIM, TBS), lambda i: (0, i)),
        out_shape=jax.ShapeDtypeStruct((OUT_DIM, B), jnp.float32),
    )(feats)


def _tc_emb_t(out_t, wide):
    def body(_, wide_ref, out_ref):
        out_ref[...] = wide_ref[:, 0:E_DIM].T

    return pl.pallas_call(
        body,
        grid=(CGRID,),
        in_specs=[
            pl.BlockSpec(memory_space=pl.ANY),
            pl.BlockSpec((CBS, F_DIM), lambda i: (i, 0)),
        ],
        out_specs=pl.BlockSpec((E_DIM, CBS), lambda i: (2, i)),
        out_shape=jax.ShapeDtypeStruct((OUT_DIM, B), jnp.float32),
        input_output_aliases={0: 0},
    )(out_t, wide)


def kernel(ids, feats, hop_idx, emb_W):
    n_nodes = emb_W.shape[0] - 1
    gather_ids = jnp.where(hop_idx > 0, ids,
                           jnp.full_like(ids, n_nodes)).astype(jnp.int32)
    emb128 = jnp.pad(emb_W, ((0, 0), (0, F_DIM - E_DIM)))
    wide = _sc_gather(gather_ids, emb128)
    out_t = _tc_feats_t(feats)
    out_t = _tc_emb_t(out_t, wide)
    return out_t.T
